# baseline (speedup 1.0000x reference)
import functools

import jax
import jax.numpy as jnp
import numpy as np
from jax.experimental import pallas as pl
from jax.experimental.pallas import tpu as pltpu

IN_F, OUT_F = 4096, 4096
NB = 65536
NW = 32


def _hadamard_matrix(n):
    h = np.array([[1.0]], dtype=np.float32)
    while h.shape[0] < n:
        h = np.block([[h, h], [h, -h]])
    return jnp.asarray(h / np.sqrt(np.float32(n)))


def _decode_kernel(a_ref, b_ref, tlp_ref, sc_ref, h_ref, out_ref, vs_ref):
    shl = 2 * (jax.lax.broadcasted_iota(jnp.int32, (8, 256), 0) & 7)
    shr = 16 - shl

    def ktile_body(kt, _):
        for v in range(4):
            a = a_ref[pl.ds(32 * kt + 8 * v, 8), :]
            b = b_ref[pl.ds(32 * kt + 8 * v, 8), :]
            s = ((a << shl) | (b >> shr)) & 0xFFFF
            lidx = (s >> 1) & 127
            hi8 = s >> 8
            l0, l1 = lidx[:, :128], lidx[:, 128:]
            h0, h1 = hi8[:, :128], hi8[:, 128:]

            z = jnp.zeros((8, 128), jnp.int32)
            aa = [z] * 8
            for c in range(256):
                chunk = tlp_ref[c]
                g0 = jnp.take_along_axis(chunk, l0, axis=1)
                g1 = jnp.take_along_axis(chunk, l1, axis=1)
                k = c & 3
                aa[k] = jnp.where(h0 == c, g0, aa[k])
                aa[4 + k] = jnp.where(h1 == c, g1, aa[4 + k])
            p0 = (aa[0] | aa[1]) | (aa[2] | aa[3])
            p1 = (aa[4] | aa[5]) | (aa[6] | aa[7])
            packed = jnp.concatenate([p0, p1], axis=1)
            odd = (s & 1) == 1
            bits = jnp.where(odd, packed & jnp.int32(-65536), packed << 16)
            vs_ref[pl.ds(32 * kt + 8 * v, 8), :] = (
                jax.lax.bitcast_convert_type(bits, jnp.float32))
        return 0

    jax.lax.fori_loop(0, 128, ktile_body, 0)

    hmat = h_ref[...]
    for g in range(32):
        blk = vs_ref[g * 128:(g + 1) * 128, :]
        w = jnp.dot(hmat, blk, preferred_element_type=jnp.float32)
        w = w * sc_ref[g:g + 1, :]
        out_ref[g * 128:(g + 1) * 128, :] = w.astype(jnp.bfloat16)


def _decode_wt(a_full, b_full, tlp, scales_t, hmat):
    grid = (2, 8)

    def idx_ab(core, n):
        return (0, core * 8 + n)

    return pl.pallas_call(
        _decode_kernel,
        grid=grid,
        in_specs=[
            pl.BlockSpec((4096, 256), idx_ab),
            pl.BlockSpec((4096, 256), idx_ab),
            pl.BlockSpec((256, 8, 128), lambda core, n: (0, 0, 0)),
            pl.BlockSpec((32, 256), idx_ab),
            pl.BlockSpec((128, 128), lambda core, n: (0, 0)),
        ],
        out_specs=pl.BlockSpec((4096, 256), idx_ab),
        out_shape=jax.ShapeDtypeStruct((4096, 4096), jnp.bfloat16),
        scratch_shapes=[pltpu.VMEM((4096, 256), jnp.float32)],
        compiler_params=pltpu.CompilerParams(
            dimension_semantics=("parallel", "arbitrary"),
            vmem_limit_bytes=100 * 1024 * 1024,
        ),
    )(a_full, b_full, tlp, scales_t, hmat)


def _matmul_kernel(x_ref, w_ref, o_ref):
    o_ref[...] = jnp.dot(x_ref[...], w_ref[...],
                         preferred_element_type=jnp.float32)


def _matmul(x16, wt):
    grid = (2, 4, 4)
    return pl.pallas_call(
        _matmul_kernel,
        grid=grid,
        in_specs=[
            pl.BlockSpec((1024, 4096), lambda c, m, n: (c * 4 + m, 0)),
            pl.BlockSpec((4096, 1024), lambda c, m, n: (0, n)),
        ],
        out_specs=pl.BlockSpec((1024, 1024), lambda c, m, n: (c * 4 + m, n)),
        out_shape=jax.ShapeDtypeStruct((8192, 4096), jnp.float32),
        compiler_params=pltpu.CompilerParams(
            dimension_semantics=("parallel", "arbitrary", "arbitrary"),
            vmem_limit_bytes=100 * 1024 * 1024,
        ),
    )(x16, wt)


def kernel(input, trellis, tlut, scales):
    def expand(t):
        t8 = t.reshape(256, 256, 16, 2).transpose(1, 3, 0, 2).reshape(512, 4096)
        return jnp.repeat(t8, 8, axis=0)

    a_full = expand(trellis)
    b_full = expand(jnp.roll(trellis, -1, axis=1))

    t16 = jax.lax.bitcast_convert_type(
        tlut[:, 0].astype(jnp.bfloat16), jnp.uint16).astype(jnp.int32)
    tp = (t16[0::2] | (t16[1::2] << 16)).reshape(256, 1, 128)
    tlp = jnp.broadcast_to(tp, (256, 8, 128))

    scales_t = scales.reshape(4096, 32).T
    hmat = _hadamard_matrix(128)

    wt = _decode_wt(a_full, b_full, tlp, scales_t, hmat)

    x16 = input.reshape(8192, 4096).astype(jnp.bfloat16)
    out = _matmul(x16, wt)
    return out.reshape(input.shape)


# speedup vs baseline: 1.3233x; 1.3233x over previous
import functools

import jax
import jax.numpy as jnp
import numpy as np
from jax.experimental import pallas as pl
from jax.experimental.pallas import tpu as pltpu

IN_F, OUT_F = 4096, 4096
NB = 65536
NW = 32


def _hadamard_matrix(n):
    h = np.array([[1.0]], dtype=np.float32)
    while h.shape[0] < n:
        h = np.block([[h, h], [h, -h]])
    return jnp.asarray(h / np.sqrt(np.float32(n)))


def _decode_kernel(a_ref, b_ref, tlp_ref, sc_ref, h_ref, out_ref, vs_ref):
    shl = 2 * (jax.lax.broadcasted_iota(jnp.int32, (8, 256), 0) & 7)
    shr = 16 - shl

    def ktile_body(kt, _):
        for v in range(4):
            a = a_ref[pl.ds(32 * kt + 8 * v, 8), :]
            b = b_ref[pl.ds(32 * kt + 8 * v, 8), :]
            s = ((a << shl) | (b >> shr)) & 0xFFFF
            lidx = (s >> 1) & 127
            hi8 = s >> 8
            l0, l1 = lidx[:, :128], lidx[:, 128:]
            h0, h1 = hi8[:, :128], hi8[:, 128:]

            def chunk_group(cg, accs, l0=l0, l1=l1, h0=h0, h1=h1):
                aa = list(accs)
                base = 128 * cg
                for u in range(128):
                    c = base + u
                    chunk = tlp_ref[c]
                    g0 = jnp.take_along_axis(chunk, l0, axis=1)
                    g1 = jnp.take_along_axis(chunk, l1, axis=1)
                    k = u & 7
                    aa[k] = jnp.where(h0 == c, g0, aa[k])
                    aa[8 + k] = jnp.where(h1 == c, g1, aa[8 + k])
                return tuple(aa)

            z = jnp.zeros((8, 128), jnp.int32)
            aa = jax.lax.fori_loop(0, 2, chunk_group, (z,) * 16)
            p0 = ((aa[0] | aa[1]) | (aa[2] | aa[3])) | (
                (aa[4] | aa[5]) | (aa[6] | aa[7]))
            p1 = ((aa[8] | aa[9]) | (aa[10] | aa[11])) | (
                (aa[12] | aa[13]) | (aa[14] | aa[15]))
            packed = jnp.concatenate([p0, p1], axis=1)
            odd = (s & 1) == 1
            bits = jnp.where(odd, packed & jnp.int32(-65536), packed << 16)
            vs_ref[pl.ds(32 * kt + 8 * v, 8), :] = (
                jax.lax.bitcast_convert_type(bits, jnp.float32))
        return 0

    jax.lax.fori_loop(0, 128, ktile_body, 0)

    hmat = h_ref[...]
    for g in range(32):
        blk = vs_ref[g * 128:(g + 1) * 128, :]
        w = jnp.dot(hmat, blk, preferred_element_type=jnp.float32)
        w = w * sc_ref[g:g + 1, :]
        out_ref[g * 128:(g + 1) * 128, :] = w.astype(jnp.bfloat16)


def _decode_wt(a_full, b_full, tlp, scales_t, hmat):
    grid = (2, 8)

    def idx_ab(core, n):
        return (0, core * 8 + n)

    return pl.pallas_call(
        _decode_kernel,
        grid=grid,
        in_specs=[
            pl.BlockSpec((4096, 256), idx_ab),
            pl.BlockSpec((4096, 256), idx_ab),
            pl.BlockSpec((256, 8, 128), lambda core, n: (0, 0, 0)),
            pl.BlockSpec((32, 256), idx_ab),
            pl.BlockSpec((128, 128), lambda core, n: (0, 0)),
        ],
        out_specs=pl.BlockSpec((4096, 256), idx_ab),
        out_shape=jax.ShapeDtypeStruct((4096, 4096), jnp.bfloat16),
        scratch_shapes=[pltpu.VMEM((4096, 256), jnp.float32)],
        compiler_params=pltpu.CompilerParams(
            dimension_semantics=("parallel", "arbitrary"),
            vmem_limit_bytes=100 * 1024 * 1024,
        ),
    )(a_full, b_full, tlp, scales_t, hmat)


def _matmul_kernel(x_ref, w_ref, o_ref):
    o_ref[...] = jnp.dot(x_ref[...], w_ref[...],
                         preferred_element_type=jnp.float32)


def _matmul(x16, wt):
    grid = (2, 4, 4)
    return pl.pallas_call(
        _matmul_kernel,
        grid=grid,
        in_specs=[
            pl.BlockSpec((1024, 4096), lambda c, m, n: (c * 4 + m, 0)),
            pl.BlockSpec((4096, 1024), lambda c, m, n: (0, n)),
        ],
        out_specs=pl.BlockSpec((1024, 1024), lambda c, m, n: (c * 4 + m, n)),
        out_shape=jax.ShapeDtypeStruct((8192, 4096), jnp.float32),
        compiler_params=pltpu.CompilerParams(
            dimension_semantics=("parallel", "arbitrary", "arbitrary"),
            vmem_limit_bytes=100 * 1024 * 1024,
        ),
    )(x16, wt)


def kernel(input, trellis, tlut, scales):
    def expand(t):
        t8 = t.reshape(256, 256, 16, 2).transpose(1, 3, 0, 2).reshape(512, 4096)
        return jnp.repeat(t8, 8, axis=0)

    a_full = expand(trellis)
    b_full = expand(jnp.roll(trellis, -1, axis=1))

    t16 = jax.lax.bitcast_convert_type(
        tlut[:, 0].astype(jnp.bfloat16), jnp.uint16).astype(jnp.int32)
    tp = (t16[0::2] | (t16[1::2] << 16)).reshape(256, 1, 128)
    tlp = jnp.broadcast_to(tp, (256, 8, 128))

    scales_t = scales.reshape(4096, 32).T
    hmat = _hadamard_matrix(128)

    wt = _decode_wt(a_full, b_full, tlp, scales_t, hmat)

    x16 = input.reshape(8192, 4096).astype(jnp.bfloat16)
    out = _matmul(x16, wt)
    return out.reshape(input.shape)


# speedup vs baseline: 1.3337x; 1.0079x over previous
import functools

import jax
import jax.numpy as jnp
import numpy as np
from jax.experimental import pallas as pl
from jax.experimental.pallas import tpu as pltpu

IN_F, OUT_F = 4096, 4096
NB = 65536
NW = 32


def _hadamard_matrix(n):
    h = np.array([[1.0]], dtype=np.float32)
    while h.shape[0] < n:
        h = np.block([[h, h], [h, -h]])
    return jnp.asarray(h / np.sqrt(np.float32(n)))


def _decode_kernel(a_ref, b_ref, tlp_ref, sc_ref, h_ref, out_ref, vs_ref):
    shl = 2 * (jax.lax.broadcasted_iota(jnp.int32, (8, 256), 0) & 7)
    shr = 16 - shl

    def ktile_body(kt, _):
        for v in range(4):
            a = jnp.broadcast_to(a_ref[pl.ds(4 * kt + v, 1), :], (8, 256))
            b = jnp.broadcast_to(b_ref[pl.ds(4 * kt + v, 1), :], (8, 256))
            s = ((a << shl) | (b >> shr)) & 0xFFFF
            lidx = (s >> 1) & 127
            hi8 = s >> 8
            l0, l1 = lidx[:, :128], lidx[:, 128:]
            h0, h1 = hi8[:, :128], hi8[:, 128:]

            def chunk_group(cg, accs, l0=l0, l1=l1, h0=h0, h1=h1):
                aa = list(accs)
                base = 128 * cg
                for u in range(128):
                    c = base + u
                    chunk = tlp_ref[c]
                    g0 = jnp.take_along_axis(chunk, l0, axis=1)
                    g1 = jnp.take_along_axis(chunk, l1, axis=1)
                    k = u & 7
                    aa[k] = jnp.where(h0 == c, g0, aa[k])
                    aa[8 + k] = jnp.where(h1 == c, g1, aa[8 + k])
                return tuple(aa)

            z = jnp.zeros((8, 128), jnp.int32)
            aa = jax.lax.fori_loop(0, 2, chunk_group, (z,) * 16)
            p0 = ((aa[0] | aa[1]) | (aa[2] | aa[3])) | (
                (aa[4] | aa[5]) | (aa[6] | aa[7]))
            p1 = ((aa[8] | aa[9]) | (aa[10] | aa[11])) | (
                (aa[12] | aa[13]) | (aa[14] | aa[15]))
            packed = jnp.concatenate([p0, p1], axis=1)
            odd = (s & 1) == 1
            bits = jnp.where(odd, packed & jnp.int32(-65536), packed << 16)
            vs_ref[pl.ds(32 * kt + 8 * v, 8), :] = (
                jax.lax.bitcast_convert_type(bits, jnp.float32))
        return 0

    jax.lax.fori_loop(0, 128, ktile_body, 0)

    hmat = h_ref[...]
    for g in range(32):
        blk = vs_ref[g * 128:(g + 1) * 128, :]
        w = jnp.dot(hmat, blk, preferred_element_type=jnp.float32)
        w = w * sc_ref[g:g + 1, :]
        out_ref[g * 128:(g + 1) * 128, :] = w.astype(jnp.bfloat16)


def _decode_wt(a_full, b_full, tlp, scales_t, hmat):
    grid = (2, 8)

    def idx_ab(core, n):
        return (0, core * 8 + n)

    return pl.pallas_call(
        _decode_kernel,
        grid=grid,
        in_specs=[
            pl.BlockSpec((512, 256), idx_ab),
            pl.BlockSpec((512, 256), idx_ab),
            pl.BlockSpec((256, 8, 128), lambda core, n: (0, 0, 0)),
            pl.BlockSpec((32, 256), idx_ab),
            pl.BlockSpec((128, 128), lambda core, n: (0, 0)),
        ],
        out_specs=pl.BlockSpec((4096, 256), idx_ab),
        out_shape=jax.ShapeDtypeStruct((4096, 4096), jnp.bfloat16),
        scratch_shapes=[pltpu.VMEM((4096, 256), jnp.float32)],
        compiler_params=pltpu.CompilerParams(
            dimension_semantics=("parallel", "arbitrary"),
            vmem_limit_bytes=100 * 1024 * 1024,
        ),
    )(a_full, b_full, tlp, scales_t, hmat)


def _matmul_kernel(x_ref, w_ref, o_ref):
    o_ref[...] = jnp.dot(x_ref[...], w_ref[...],
                         preferred_element_type=jnp.float32)


def _matmul(x16, wt):
    grid = (2, 4, 4)
    return pl.pallas_call(
        _matmul_kernel,
        grid=grid,
        in_specs=[
            pl.BlockSpec((1024, 4096), lambda c, m, n: (c * 4 + m, 0)),
            pl.BlockSpec((4096, 1024), lambda c, m, n: (0, n)),
        ],
        out_specs=pl.BlockSpec((1024, 1024), lambda c, m, n: (c * 4 + m, n)),
        out_shape=jax.ShapeDtypeStruct((8192, 4096), jnp.float32),
        compiler_params=pltpu.CompilerParams(
            dimension_semantics=("parallel", "arbitrary", "arbitrary"),
            vmem_limit_bytes=100 * 1024 * 1024,
        ),
    )(x16, wt)


def kernel(input, trellis, tlut, scales):
    def expand(t):
        return t.reshape(256, 256, 16, 2).transpose(1, 3, 0, 2).reshape(512, 4096)

    a_full = expand(trellis)
    b_full = expand(jnp.roll(trellis, -1, axis=1))

    t16 = jax.lax.bitcast_convert_type(
        tlut[:, 0].astype(jnp.bfloat16), jnp.uint16).astype(jnp.int32)
    tp = (t16[0::2] | (t16[1::2] << 16)).reshape(256, 1, 128)
    tlp = jnp.broadcast_to(tp, (256, 8, 128))

    scales_t = scales.reshape(4096, 32).T
    hmat = _hadamard_matrix(128)

    wt = _decode_wt(a_full, b_full, tlp, scales_t, hmat)

    x16 = input.reshape(8192, 4096).astype(jnp.bfloat16)
    out = _matmul(x16, wt)
    return out.reshape(input.shape)


# speedup vs baseline: 1.3454x; 1.0087x over previous
import functools

import jax
import jax.numpy as jnp
import numpy as np
from jax.experimental import pallas as pl
from jax.experimental.pallas import tpu as pltpu

IN_F, OUT_F = 4096, 4096
NB = 65536
NW = 32


def _hadamard_matrix(n):
    h = np.array([[1.0]], dtype=np.float32)
    while h.shape[0] < n:
        h = np.block([[h, h], [h, -h]])
    return jnp.asarray(h / np.sqrt(np.float32(n)))


def _decode_kernel(a_ref, b_ref, tlp_ref, sc_ref, h_ref, out_ref, vs_ref):
    shl = 2 * (jax.lax.broadcasted_iota(jnp.int32, (8, 256), 0) & 7)
    shr = 16 - shl

    def ktile_body(kt, _):
        for v in range(4):
            a = jnp.broadcast_to(a_ref[pl.ds(4 * kt + v, 1), :], (8, 256))
            b = jnp.broadcast_to(b_ref[pl.ds(4 * kt + v, 1), :], (8, 256))
            s = ((a << shl) | (b >> shr)) & 0xFFFF
            lidx = (s >> 1) & 127
            hi8 = s >> 8
            l0, l1 = lidx[:, :128], lidx[:, 128:]
            h0, h1 = hi8[:, :128], hi8[:, 128:]

            def chunk_group(cg, accs, l0=l0, l1=l1, h0=h0, h1=h1):
                aa = list(accs)
                base = 128 * cg
                for u in range(128):
                    c = base + u
                    chunk = tlp_ref[c]
                    g0 = jnp.take_along_axis(chunk, l0, axis=1)
                    g1 = jnp.take_along_axis(chunk, l1, axis=1)
                    k = u & 3
                    aa[k] = jnp.where(h0 == c, g0, aa[k])
                    aa[4 + k] = jnp.where(h1 == c, g1, aa[4 + k])
                return tuple(aa)

            z = jnp.zeros((8, 128), jnp.int32)
            aa = jax.lax.fori_loop(0, 2, chunk_group, (z,) * 8)
            p0 = (aa[0] | aa[1]) | (aa[2] | aa[3])
            p1 = (aa[4] | aa[5]) | (aa[6] | aa[7])
            packed = jnp.concatenate([p0, p1], axis=1)
            odd = (s & 1) == 1
            bits = jnp.where(odd, packed & jnp.int32(-65536), packed << 16)
            vs_ref[pl.ds(32 * kt + 8 * v, 8), :] = (
                jax.lax.bitcast_convert_type(bits, jnp.float32))
        return 0

    jax.lax.fori_loop(0, 128, ktile_body, 0)

    hmat = h_ref[...]
    for g in range(32):
        blk = vs_ref[g * 128:(g + 1) * 128, :]
        w = jnp.dot(hmat, blk, preferred_element_type=jnp.float32)
        w = w * sc_ref[g:g + 1, :]
        out_ref[g * 128:(g + 1) * 128, :] = w.astype(jnp.bfloat16)


def _decode_wt(a_full, b_full, tlp, scales_t, hmat):
    grid = (2, 8)

    def idx_ab(core, n):
        return (0, core * 8 + n)

    return pl.pallas_call(
        _decode_kernel,
        grid=grid,
        in_specs=[
            pl.BlockSpec((512, 256), idx_ab),
            pl.BlockSpec((512, 256), idx_ab),
            pl.BlockSpec((256, 8, 128), lambda core, n: (0, 0, 0)),
            pl.BlockSpec((32, 256), idx_ab),
            pl.BlockSpec((128, 128), lambda core, n: (0, 0)),
        ],
        out_specs=pl.BlockSpec((4096, 256), idx_ab),
        out_shape=jax.ShapeDtypeStruct((4096, 4096), jnp.bfloat16),
        scratch_shapes=[pltpu.VMEM((4096, 256), jnp.float32)],
        compiler_params=pltpu.CompilerParams(
            dimension_semantics=("parallel", "arbitrary"),
            vmem_limit_bytes=100 * 1024 * 1024,
        ),
    )(a_full, b_full, tlp, scales_t, hmat)


def _matmul_kernel(x_ref, w_ref, o_ref):
    o_ref[...] = jnp.dot(x_ref[...], w_ref[...],
                         preferred_element_type=jnp.float32)


def _matmul(x16, wt):
    grid = (2, 4, 4)
    return pl.pallas_call(
        _matmul_kernel,
        grid=grid,
        in_specs=[
            pl.BlockSpec((1024, 4096), lambda c, m, n: (c * 4 + m, 0)),
            pl.BlockSpec((4096, 1024), lambda c, m, n: (0, n)),
        ],
        out_specs=pl.BlockSpec((1024, 1024), lambda c, m, n: (c * 4 + m, n)),
        out_shape=jax.ShapeDtypeStruct((8192, 4096), jnp.float32),
        compiler_params=pltpu.CompilerParams(
            dimension_semantics=("parallel", "arbitrary", "arbitrary"),
            vmem_limit_bytes=100 * 1024 * 1024,
        ),
    )(x16, wt)


def kernel(input, trellis, tlut, scales):
    def expand(t):
        return t.reshape(256, 256, 16, 2).transpose(1, 3, 0, 2).reshape(512, 4096)

    a_full = expand(trellis)
    b_full = expand(jnp.roll(trellis, -1, axis=1))

    t16 = jax.lax.bitcast_convert_type(
        tlut[:, 0].astype(jnp.bfloat16), jnp.uint16).astype(jnp.int32)
    tp = (t16[0::2] | (t16[1::2] << 16)).reshape(256, 1, 128)
    tlp = jnp.broadcast_to(tp, (256, 8, 128))

    scales_t = scales.reshape(4096, 32).T
    hmat = _hadamard_matrix(128)

    wt = _decode_wt(a_full, b_full, tlp, scales_t, hmat)

    x16 = input.reshape(8192, 4096).astype(jnp.bfloat16)
    out = _matmul(x16, wt)
    return out.reshape(input.shape)


# speedup vs baseline: 1.4381x; 1.0689x over previous
import functools

import jax
import jax.numpy as jnp
import numpy as np
from jax.experimental import pallas as pl
from jax.experimental.pallas import tpu as pltpu

IN_F, OUT_F = 4096, 4096
NB = 65536
NW = 32


def _hadamard_matrix(n):
    h = np.array([[1.0]], dtype=np.float32)
    while h.shape[0] < n:
        h = np.block([[h, h], [h, -h]])
    return jnp.asarray(h / np.sqrt(np.float32(n)))


def _decode_kernel(a_ref, b_ref, tlp_ref, sc_ref, h_ref, out_ref, vs_ref):
    shl = 2 * (jax.lax.broadcasted_iota(jnp.int32, (8, 256), 0) & 7)
    shr = 16 - shl

    def pair_body(i, _):
        a = jnp.broadcast_to(a_ref[pl.ds(i, 1), :], (8, 256))
        b = jnp.broadcast_to(b_ref[pl.ds(i, 1), :], (8, 256))
        s = ((a << shl) | (b >> shr)) & 0xFFFF
        lidx = (s >> 1) & 127
        hi8 = s >> 8
        l0, l1 = lidx[:, :128], lidx[:, 128:]
        h0, h1 = hi8[:, :128], hi8[:, 128:]

        z = jnp.zeros((8, 128), jnp.int32)
        aa = [z] * 8
        for c in range(256):
            chunk = tlp_ref[c]
            g0 = jnp.take_along_axis(chunk, l0, axis=1)
            g1 = jnp.take_along_axis(chunk, l1, axis=1)
            k = c & 3
            aa[k] = jnp.where(h0 == c, g0, aa[k])
            aa[4 + k] = jnp.where(h1 == c, g1, aa[4 + k])
        p0 = (aa[0] | aa[1]) | (aa[2] | aa[3])
        p1 = (aa[4] | aa[5]) | (aa[6] | aa[7])
        packed = jnp.concatenate([p0, p1], axis=1)
        odd = (s & 1) == 1
        bits = jnp.where(odd, packed & jnp.int32(-65536), packed << 16)
        vs_ref[pl.ds(8 * i, 8), :] = jax.lax.bitcast_convert_type(
            bits, jnp.float32)
        return 0

    jax.lax.fori_loop(0, 512, pair_body, 0)

    hmat = h_ref[...]
    for g in range(32):
        blk = vs_ref[g * 128:(g + 1) * 128, :]
        w = jnp.dot(hmat, blk, preferred_element_type=jnp.float32)
        w = w * sc_ref[g:g + 1, :]
        out_ref[g * 128:(g + 1) * 128, :] = w.astype(jnp.bfloat16)


def _decode_wt(a_full, b_full, tlp, scales_t, hmat):
    grid = (2, 8)

    def idx_ab(core, n):
        return (0, core * 8 + n)

    return pl.pallas_call(
        _decode_kernel,
        grid=grid,
        in_specs=[
            pl.BlockSpec((512, 256), idx_ab),
            pl.BlockSpec((512, 256), idx_ab),
            pl.BlockSpec((256, 8, 128), lambda core, n: (0, 0, 0)),
            pl.BlockSpec((32, 256), idx_ab),
            pl.BlockSpec((128, 128), lambda core, n: (0, 0)),
        ],
        out_specs=pl.BlockSpec((4096, 256), idx_ab),
        out_shape=jax.ShapeDtypeStruct((4096, 4096), jnp.bfloat16),
        scratch_shapes=[pltpu.VMEM((4096, 256), jnp.float32)],
        compiler_params=pltpu.CompilerParams(
            dimension_semantics=("parallel", "arbitrary"),
            vmem_limit_bytes=100 * 1024 * 1024,
        ),
    )(a_full, b_full, tlp, scales_t, hmat)


def _matmul_kernel(x_ref, w_ref, o_ref):
    o_ref[...] = jnp.dot(x_ref[...], w_ref[...],
                         preferred_element_type=jnp.float32)


def _matmul(x16, wt):
    grid = (2, 4, 4)
    return pl.pallas_call(
        _matmul_kernel,
        grid=grid,
        in_specs=[
            pl.BlockSpec((1024, 4096), lambda c, m, n: (c * 4 + m, 0)),
            pl.BlockSpec((4096, 1024), lambda c, m, n: (0, n)),
        ],
        out_specs=pl.BlockSpec((1024, 1024), lambda c, m, n: (c * 4 + m, n)),
        out_shape=jax.ShapeDtypeStruct((8192, 4096), jnp.float32),
        compiler_params=pltpu.CompilerParams(
            dimension_semantics=("parallel", "arbitrary", "arbitrary"),
            vmem_limit_bytes=100 * 1024 * 1024,
        ),
    )(x16, wt)


def kernel(input, trellis, tlut, scales):
    def expand(t):
        return t.reshape(256, 256, 16, 2).transpose(1, 3, 0, 2).reshape(512, 4096)

    a_full = expand(trellis)
    b_full = expand(jnp.roll(trellis, -1, axis=1))

    t16 = jax.lax.bitcast_convert_type(
        tlut[:, 0].astype(jnp.bfloat16), jnp.uint16).astype(jnp.int32)
    tp = (t16[0::2] | (t16[1::2] << 16)).reshape(256, 1, 128)
    tlp = jnp.broadcast_to(tp, (256, 8, 128))

    scales_t = scales.reshape(4096, 32).T
    hmat = _hadamard_matrix(128)

    wt = _decode_wt(a_full, b_full, tlp, scales_t, hmat)

    x16 = input.reshape(8192, 4096).astype(jnp.bfloat16)
    out = _matmul(x16, wt)
    return out.reshape(input.shape)


# speedup vs baseline: 1.4397x; 1.0011x over previous
import jax
import jax.numpy as jnp
import numpy as np
from jax.experimental import pallas as pl
from jax.experimental.pallas import tpu as pltpu


def _hadamard_matrix(n):
    h = np.array([[1.0]], dtype=np.float32)
    while h.shape[0] < n:
        h = np.block([[h, h], [h, -h]])
    return jnp.asarray(h / np.sqrt(np.float32(n)))


def _decode_kernel(a_ref, b_ref, tlp_ref, sc_ref, h_ref, out_ref, vs_ref):
    shl = 2 * (jax.lax.broadcasted_iota(jnp.int32, (8, 256), 0) & 7)
    shr = 16 - shl

    def pair_body(i, _):
        a = jnp.broadcast_to(a_ref[pl.ds(i, 1), :], (8, 256))
        b = jnp.broadcast_to(b_ref[pl.ds(i, 1), :], (8, 256))
        s = ((a << shl) | (b >> shr)) & 0xFFFF
        lidx = (s >> 1) & 127
        hi8 = s >> 8
        l0, l1 = lidx[:, :128], lidx[:, 128:]
        h0, h1 = hi8[:, :128], hi8[:, 128:]

        z = jnp.zeros((8, 128), jnp.int32)
        aa = [z] * 8
        for c in range(256):
            chunk = tlp_ref[c]
            g0 = jnp.take_along_axis(chunk, l0, axis=1)
            g1 = jnp.take_along_axis(chunk, l1, axis=1)
            k = c & 3
            aa[k] = jnp.where(h0 == c, g0, aa[k])
            aa[4 + k] = jnp.where(h1 == c, g1, aa[4 + k])
        p0 = (aa[0] | aa[1]) | (aa[2] | aa[3])
        p1 = (aa[4] | aa[5]) | (aa[6] | aa[7])
        packed = jnp.concatenate([p0, p1], axis=1)
        odd = (s & 1) == 1
        bits = jnp.where(odd, packed & jnp.int32(-65536), packed << 16)
        vs_ref[pl.ds(8 * i, 8), :] = jax.lax.bitcast_convert_type(
            bits, jnp.float32)
        return 0

    jax.lax.fori_loop(0, 512, pair_body, 0)

    hmat = h_ref[...]
    for g in range(32):
        blk = vs_ref[g * 128:(g + 1) * 128, :]
        w = jnp.dot(hmat, blk, preferred_element_type=jnp.float32)
        w = w * sc_ref[g:g + 1, :]
        out_ref[g * 128:(g + 1) * 128, :] = w.astype(jnp.bfloat16)


def _decode_wt(a_full, b_full, tlp, scales_t, hmat):
    grid = (2, 8)

    def idx_ab(core, n):
        return (0, core * 8 + n)

    return pl.pallas_call(
        _decode_kernel,
        grid=grid,
        in_specs=[
            pl.BlockSpec((512, 256), idx_ab),
            pl.BlockSpec((512, 256), idx_ab),
            pl.BlockSpec((256, 8, 128), lambda core, n: (0, 0, 0)),
            pl.BlockSpec((32, 256), idx_ab),
            pl.BlockSpec((128, 128), lambda core, n: (0, 0)),
        ],
        out_specs=pl.BlockSpec((4096, 256), idx_ab),
        out_shape=jax.ShapeDtypeStruct((4096, 4096), jnp.bfloat16),
        scratch_shapes=[pltpu.VMEM((4096, 256), jnp.float32)],
        compiler_params=pltpu.CompilerParams(
            dimension_semantics=("parallel", "arbitrary"),
            vmem_limit_bytes=100 * 1024 * 1024,
        ),
    )(a_full, b_full, tlp, scales_t, hmat)


def _matmul_kernel(x_ref, w_ref, o_ref):
    o_ref[...] = jnp.dot(x_ref[...], w_ref[...],
                         preferred_element_type=jnp.float32)


def _matmul(x16, wt):
    grid = (2, 4, 4)
    return pl.pallas_call(
        _matmul_kernel,
        grid=grid,
        in_specs=[
            pl.BlockSpec((1024, 4096), lambda c, m, n: (c * 4 + m, 0)),
            pl.BlockSpec((4096, 1024), lambda c, m, n: (0, n)),
        ],
        out_specs=pl.BlockSpec((1024, 1024), lambda c, m, n: (c * 4 + m, n)),
        out_shape=jax.ShapeDtypeStruct((8192, 4096), jnp.float32),
        compiler_params=pltpu.CompilerParams(
            dimension_semantics=("parallel", "arbitrary", "arbitrary"),
            vmem_limit_bytes=100 * 1024 * 1024,
        ),
    )(x16, wt)


def kernel(input, trellis, tlut, scales):
    def expand(t):
        return t.reshape(256, 256, 16, 2).transpose(1, 3, 0, 2).reshape(512, 4096)

    a_full = expand(trellis)
    b_full = expand(jnp.roll(trellis, -1, axis=1))

    t16 = jax.lax.bitcast_convert_type(
        tlut[:, 0].astype(jnp.bfloat16), jnp.uint16).astype(jnp.int32)
    tp = (t16[0::2] | (t16[1::2] << 16)).reshape(256, 1, 128)
    tlp = jnp.broadcast_to(tp, (256, 8, 128))

    scales_t = scales.reshape(4096, 32).T
    hmat = _hadamard_matrix(128)

    wt = _decode_wt(a_full, b_full, tlp, scales_t, hmat)

    x16 = input.reshape(8192, 4096).astype(jnp.bfloat16)
    out = _matmul(x16, wt)
    return out.reshape(input.shape)
